# Initial kernel scaffold; baseline (speedup 1.0000x reference)
import jax
import jax.numpy as jnp
from jax.experimental import pallas as pl


def kernel(node_embeddings, edge_index, batch, W1, b1, ln1_g, ln1_b, W2, b2, ln2_g, ln2_b):
    raise NotImplementedError("write your pallas kernel here")


# speedup vs baseline: 4.4055x; 4.4055x over previous
import functools

import jax
import jax.numpy as jnp
from jax import lax
from jax.experimental import pallas as pl
from jax.experimental.pallas import tpu as pltpu
from jax.experimental.pallas import tpu_sc as plsc

B = 2048
H = 8
D = 128
HD = 128
SUB = 16
ROWS = H * SUB

NC = 2
NS = 16
NW = NC * NS
GPT = B // NW


def _pool_block(buf, stage):
    for h in range(H):
        for c in range(D // 16):
            col = pl.ds(16 * c, 16)
            s = buf[16 * h, col]
            m = s
            for r in range(1, SUB):
                v = buf[16 * h + r, col]
                s = s + v
                m = jnp.maximum(m, v)
            base = h * (3 * D) + 16 * c
            stage[0, pl.ds(base, 16)] = s * (1.0 / SUB)
            stage[0, pl.ds(base + D, 16)] = m
            stage[0, pl.ds(base + 2 * D, 16)] = s


def _sc_gather_pool(emb, idx):
    mesh = plsc.VectorSubcoreMesh(
        core_axis_name="c", subcore_axis_name="s",
        num_cores=NC, num_subcores=NS)

    @functools.partial(
        pl.kernel,
        out_type=jax.ShapeDtypeStruct((B, 3 * H * D), jnp.float32),
        mesh=mesh,
        scratch_types=[
            pltpu.VMEM((GPT, ROWS), jnp.int32),
            pltpu.VMEM((ROWS, D), jnp.float32),
            pltpu.VMEM((ROWS, D), jnp.float32),
            pltpu.VMEM((1, 3 * H * D), jnp.float32),
            pltpu.VMEM((1, 3 * H * D), jnp.float32),
            pltpu.SemaphoreType.DMA,
            pltpu.SemaphoreType.DMA,
            pltpu.SemaphoreType.DMA,
            pltpu.SemaphoreType.DMA,
        ],
    )
    def k(emb_hbm, idx_hbm, out_hbm, idx_v, buf0, buf1, st0, st1,
          sg0, sg1, so0, so1):
        wid = lax.axis_index("s") * NC + lax.axis_index("c")
        base_row = wid * GPT
        pltpu.sync_copy(idx_hbm.at[wid], idx_v)

        def gather_start(g, buf, sem):
            pltpu.make_async_copy(emb_hbm.at[idx_v.at[g]], buf, sem).start()

        def gather_wait(buf, sem):
            pltpu.make_async_copy(emb_hbm.at[idx_v.at[0]], buf, sem).wait()

        def out_start(g, st, sem):
            pltpu.make_async_copy(
                st, out_hbm.at[pl.ds(base_row + g, 1)], sem).start()

        def out_wait(st, sem):
            pltpu.make_async_copy(
                st, out_hbm.at[pl.ds(base_row, 1)], sem).wait()

        gather_start(0, buf0, sg0)

        def body(i2, carry):
            g = 2 * i2
            gather_start(g + 1, buf1, sg1)
            gather_wait(buf0, sg0)

            @pl.when(i2 > 0)
            def _():
                out_wait(st0, so0)

            _pool_block(buf0, st0)
            out_start(g, st0, so0)

            @pl.when(i2 < GPT // 2 - 1)
            def _():
                gather_start(g + 2, buf0, sg0)

            gather_wait(buf1, sg1)

            @pl.when(i2 > 0)
            def _():
                out_wait(st1, so1)

            _pool_block(buf1, st1)
            out_start(g + 1, st1, so1)
            return carry

        lax.fori_loop(0, GPT // 2, body, 0)
        out_wait(st0, so0)
        out_wait(st1, so1)

    return k(emb, idx)


def _mlp_body(x_ref, w1_ref, b1_ref, g1_ref, bb1_ref,
              w2_ref, b2_ref, g2_ref, bb2_ref, o_ref):
    x = x_ref[...]
    h1 = jnp.dot(x, w1_ref[0], preferred_element_type=jnp.float32)
    h1 = h1 + b1_ref[0]
    mu = jnp.mean(h1, axis=-1, keepdims=True)
    var = jnp.mean((h1 - mu) ** 2, axis=-1, keepdims=True)
    h1 = (h1 - mu) / jnp.sqrt(var + 1e-5) * g1_ref[...] + bb1_ref[...]
    h1 = jnp.maximum(h1, 0.0)
    o = jnp.dot(h1, w2_ref[0], preferred_element_type=jnp.float32)
    o = o + b2_ref[0]
    mu2 = jnp.mean(o, axis=-1, keepdims=True)
    var2 = jnp.mean((o - mu2) ** 2, axis=-1, keepdims=True)
    o = (o - mu2) / jnp.sqrt(var2 + 1e-5) * g2_ref[...] + bb2_ref[...]
    o_ref[...] = jnp.maximum(o, 0.0)


def _mlp(comb, W1, b1, ln1_g, ln1_b, W2, b2, ln2_g, ln2_b):
    Bblk = 256
    grid = (H, B // Bblk)
    return pl.pallas_call(
        _mlp_body,
        grid=grid,
        in_specs=[
            pl.BlockSpec((Bblk, 3 * D), lambda h, i: (i, h)),
            pl.BlockSpec((1, 3 * D, 2 * HD), lambda h, i: (h, 0, 0)),
            pl.BlockSpec((1, 1, 2 * HD), lambda h, i: (h, 0, 0)),
            pl.BlockSpec((1, 2 * HD), lambda h, i: (0, 0)),
            pl.BlockSpec((1, 2 * HD), lambda h, i: (0, 0)),
            pl.BlockSpec((1, 2 * HD, HD), lambda h, i: (h, 0, 0)),
            pl.BlockSpec((1, 1, HD), lambda h, i: (h, 0, 0)),
            pl.BlockSpec((1, HD), lambda h, i: (0, 0)),
            pl.BlockSpec((1, HD), lambda h, i: (0, 0)),
        ],
        out_specs=pl.BlockSpec((Bblk, HD), lambda h, i: (i, h)),
        out_shape=jax.ShapeDtypeStruct((B, H * HD), jnp.float32),
    )(comb, W1, b1.reshape(H, 1, 2 * HD), ln1_g.reshape(1, -1),
      ln1_b.reshape(1, -1), W2, b2.reshape(H, 1, HD),
      ln2_g.reshape(1, -1), ln2_b.reshape(1, -1))


def kernel(node_embeddings, edge_index, batch, W1, b1, ln1_g, ln1_b,
           W2, b2, ln2_g, ln2_b):
    del edge_index
    n = node_embeddings.shape[0]
    offsets = jnp.searchsorted(
        batch, jnp.arange(B, dtype=batch.dtype)).astype(jnp.int32)
    idx = jnp.minimum(
        offsets[:, None] + jnp.arange(ROWS, dtype=jnp.int32)[None, :], n - 1)
    idx = idx.reshape(NW, GPT, ROWS)
    comb = _sc_gather_pool(node_embeddings, idx)
    out = _mlp(comb, W1, b1, ln1_g, ln1_b, W2, b2, ln2_g, ln2_b)
    return out.reshape(B, H, HD)


# speedup vs baseline: 5.4352x; 1.2337x over previous
import functools

import jax
import jax.numpy as jnp
from jax import lax
from jax.experimental import pallas as pl
from jax.experimental.pallas import tpu as pltpu
from jax.experimental.pallas import tpu_sc as plsc

B = 2048
H = 8
D = 128
HD = 128
SUB = 16
ROWS = H * SUB

NC = 2
NS = 16
NW = NC * NS
GPT = B // NW


def _pool_block(buf, stage):
    def hbody(h, carry):
        row0 = SUB * h
        for c in range(D // 16):
            col = pl.ds(16 * c, 16)
            s = buf[row0, col]
            m = s
            for r in range(1, SUB):
                v = buf[row0 + r, col]
                s = s + v
                m = jnp.maximum(m, v)
            base = h * (3 * D) + 16 * c
            stage[0, pl.ds(base, 16)] = s * (1.0 / SUB)
            stage[0, pl.ds(base + D, 16)] = m
            stage[0, pl.ds(base + 2 * D, 16)] = s
        return carry

    lax.fori_loop(0, H, hbody, 0)


def _sc_gather_pool(emb, idx):
    mesh = plsc.VectorSubcoreMesh(
        core_axis_name="c", subcore_axis_name="s",
        num_cores=NC, num_subcores=NS)

    @functools.partial(
        pl.kernel,
        out_type=jax.ShapeDtypeStruct((B, 3 * H * D), jnp.float32),
        mesh=mesh,
        scratch_types=[
            pltpu.VMEM((GPT, ROWS), jnp.int32),
            pltpu.VMEM((ROWS, D), jnp.float32),
            pltpu.VMEM((ROWS, D), jnp.float32),
            pltpu.VMEM((1, 3 * H * D), jnp.float32),
            pltpu.VMEM((1, 3 * H * D), jnp.float32),
            pltpu.SemaphoreType.DMA,
            pltpu.SemaphoreType.DMA,
            pltpu.SemaphoreType.DMA,
            pltpu.SemaphoreType.DMA,
        ],
    )
    def k(emb_hbm, idx_hbm, out_hbm, idx_v, buf0, buf1, st0, st1,
          sg0, sg1, so0, so1):
        wid = lax.axis_index("s") * NC + lax.axis_index("c")
        base_row = wid * GPT
        pltpu.sync_copy(idx_hbm.at[wid], idx_v)

        def gather_start(g, buf, sem):
            pltpu.make_async_copy(emb_hbm.at[idx_v.at[g]], buf, sem).start()

        def gather_wait(buf, sem):
            pltpu.make_async_copy(emb_hbm.at[idx_v.at[0]], buf, sem).wait()

        def out_start(g, st, sem):
            pltpu.make_async_copy(
                st, out_hbm.at[pl.ds(base_row + g, 1)], sem).start()

        def out_wait(st, sem):
            pltpu.make_async_copy(
                st, out_hbm.at[pl.ds(base_row, 1)], sem).wait()

        gather_start(0, buf0, sg0)

        def body(i2, carry):
            g = 2 * i2
            gather_start(g + 1, buf1, sg1)
            gather_wait(buf0, sg0)

            @pl.when(i2 > 0)
            def _():
                out_wait(st0, so0)

            _pool_block(buf0, st0)
            out_start(g, st0, so0)

            @pl.when(i2 < GPT // 2 - 1)
            def _():
                gather_start(g + 2, buf0, sg0)

            gather_wait(buf1, sg1)

            @pl.when(i2 > 0)
            def _():
                out_wait(st1, so1)

            _pool_block(buf1, st1)
            out_start(g + 1, st1, so1)
            return carry

        lax.fori_loop(0, GPT // 2, body, 0)
        out_wait(st0, so0)
        out_wait(st1, so1)

    return k(emb, idx)


def _mlp_body(x_ref, w1_ref, b1_ref, g1_ref, bb1_ref,
              w2_ref, b2_ref, g2_ref, bb2_ref, o_ref):
    x = x_ref[...]
    h1 = jnp.dot(x, w1_ref[0], preferred_element_type=jnp.float32)
    h1 = h1 + b1_ref[0]
    mu = jnp.mean(h1, axis=-1, keepdims=True)
    var = jnp.mean((h1 - mu) ** 2, axis=-1, keepdims=True)
    h1 = (h1 - mu) / jnp.sqrt(var + 1e-5) * g1_ref[...] + bb1_ref[...]
    h1 = jnp.maximum(h1, 0.0)
    o = jnp.dot(h1, w2_ref[0], preferred_element_type=jnp.float32)
    o = o + b2_ref[0]
    mu2 = jnp.mean(o, axis=-1, keepdims=True)
    var2 = jnp.mean((o - mu2) ** 2, axis=-1, keepdims=True)
    o = (o - mu2) / jnp.sqrt(var2 + 1e-5) * g2_ref[...] + bb2_ref[...]
    o_ref[...] = jnp.maximum(o, 0.0)


def _mlp(comb, W1, b1, ln1_g, ln1_b, W2, b2, ln2_g, ln2_b):
    Bblk = 256
    grid = (H, B // Bblk)
    return pl.pallas_call(
        _mlp_body,
        grid=grid,
        in_specs=[
            pl.BlockSpec((Bblk, 3 * D), lambda h, i: (i, h)),
            pl.BlockSpec((1, 3 * D, 2 * HD), lambda h, i: (h, 0, 0)),
            pl.BlockSpec((1, 1, 2 * HD), lambda h, i: (h, 0, 0)),
            pl.BlockSpec((1, 2 * HD), lambda h, i: (0, 0)),
            pl.BlockSpec((1, 2 * HD), lambda h, i: (0, 0)),
            pl.BlockSpec((1, 2 * HD, HD), lambda h, i: (h, 0, 0)),
            pl.BlockSpec((1, 1, HD), lambda h, i: (h, 0, 0)),
            pl.BlockSpec((1, HD), lambda h, i: (0, 0)),
            pl.BlockSpec((1, HD), lambda h, i: (0, 0)),
        ],
        out_specs=pl.BlockSpec((Bblk, HD), lambda h, i: (i, h)),
        out_shape=jax.ShapeDtypeStruct((B, H * HD), jnp.float32),
    )(comb, W1, b1.reshape(H, 1, 2 * HD), ln1_g.reshape(1, -1),
      ln1_b.reshape(1, -1), W2, b2.reshape(H, 1, HD),
      ln2_g.reshape(1, -1), ln2_b.reshape(1, -1))


def kernel(node_embeddings, edge_index, batch, W1, b1, ln1_g, ln1_b,
           W2, b2, ln2_g, ln2_b):
    del edge_index
    n = node_embeddings.shape[0]
    offsets = jnp.searchsorted(
        batch, jnp.arange(B, dtype=batch.dtype)).astype(jnp.int32)
    idx = jnp.minimum(
        offsets[:, None] + jnp.arange(ROWS, dtype=jnp.int32)[None, :], n - 1)
    idx = idx.reshape(NW, GPT, ROWS)
    comb = _sc_gather_pool(node_embeddings, idx)
    out = _mlp(comb, W1, b1, ln1_g, ln1_b, W2, b2, ln2_g, ln2_b)
    return out.reshape(B, H, HD)


# speedup vs baseline: 8.5817x; 1.5789x over previous
import functools

import jax
import jax.numpy as jnp
from jax import lax
from jax.experimental import pallas as pl
from jax.experimental.pallas import tpu as pltpu
from jax.experimental.pallas import tpu_sc as plsc

B = 2048
H = 8
D = 128
HD = 128
SUB = 16
ROWS = H * SUB

NC = 2
NS = 16
NW = NC * NS
GPT = B // NW


def _pool_block(buf, stage):
    def hbody(h, carry):
        row0 = SUB * h
        for c in range(D // 16):
            col = pl.ds(16 * c, 16)
            s = buf[row0, col]
            m = s
            for r in range(1, SUB):
                v = buf[row0 + r, col]
                s = s + v
                m = jnp.maximum(m, v)
            base = h * (3 * D) + 16 * c
            stage[0, pl.ds(base, 16)] = s * (1.0 / SUB)
            stage[0, pl.ds(base + D, 16)] = m
            stage[0, pl.ds(base + 2 * D, 16)] = s
        return carry

    lax.fori_loop(0, H, hbody, 0)


def _sc_gather_pool(emb, batch):
    n = emb.shape[0]
    scan = n // NS
    nvec = scan // 16
    mesh = plsc.VectorSubcoreMesh(
        core_axis_name="c", subcore_axis_name="s",
        num_cores=NC, num_subcores=NS)

    @functools.partial(
        pl.kernel,
        out_type=jax.ShapeDtypeStruct((B, 3 * H * D), jnp.float32),
        mesh=mesh,
        compiler_params=pltpu.CompilerParams(needs_layout_passes=False),
        scratch_types=[
            pltpu.VMEM((16 + scan,), jnp.int32),
            pltpu.VMEM((B,), jnp.int32),
            pltpu.VMEM((NS, B), jnp.int32),
            pltpu.VMEM((B,), jnp.int32),
            pltpu.VMEM((ROWS,), jnp.int32),
            pltpu.VMEM((ROWS,), jnp.int32),
            pltpu.VMEM((ROWS, D), jnp.float32),
            pltpu.VMEM((ROWS, D), jnp.float32),
            pltpu.VMEM((1, 3 * H * D), jnp.float32),
            pltpu.VMEM((1, 3 * H * D), jnp.float32),
            pltpu.VMEM_SHARED((NS, B), jnp.int32),
            pltpu.SemaphoreType.DMA,
            pltpu.SemaphoreType.DMA,
            pltpu.SemaphoreType.DMA,
            pltpu.SemaphoreType.DMA,
        ],
    )
    def k(emb_hbm, batch_hbm, out_hbm, chunk_v, part_v, allp_v, off_v,
          idx0, idx1, buf0, buf1, st0, st1, spmem, sg0, sg1, so0, so1):
        cid = lax.axis_index("c")
        sid = lax.axis_index("s")
        wid = sid * NC + cid
        iota = lax.iota(jnp.int32, 16)

        cstart = sid * scan
        pltpu.sync_copy(batch_hbm.at[pl.ds(cstart, scan)],
                        chunk_v.at[pl.ds(16, scan)])

        @pl.when(sid == 0)
        def _():
            chunk_v[pl.ds(0, 16)] = jnp.full((16,), -1, jnp.int32)

        @pl.when(sid > 0)
        def _():
            pltpu.sync_copy(batch_hbm.at[pl.ds(cstart - 16, 16)],
                            chunk_v.at[pl.ds(0, 16)])

        def initp(kk, carry):
            part_v[pl.ds(16 * kk, 16)] = jnp.full((16,), n, jnp.int32)
            return carry

        lax.fori_loop(0, B // 16, initp, 0)

        def scan_body(i, carry):
            base = 16 * i
            cur = chunk_v[pl.ds(16 + base, 16)]
            prev = chunk_v[pl.ds(15 + base, 16)]
            pos = jnp.full((16,), cstart + base, jnp.int32) + iota
            plsc.store_scatter(part_v, [cur], pos, mask=cur != prev)
            return carry

        lax.fori_loop(0, nvec, scan_body, 0)

        pltpu.sync_copy(part_v, spmem.at[sid])
        plsc.subcore_barrier()
        pltpu.sync_copy(spmem, allp_v)

        def comb_body(kk, carry):
            col = pl.ds(16 * kk, 16)
            m = allp_v[0, col]
            for s2 in range(1, NS):
                m = jnp.minimum(m, allp_v[s2, col])
            off_v[col] = m
            return carry

        lax.fori_loop(0, B // 16, comb_body, 0)

        def suf_body(k2, carry):
            col = pl.ds(16 * ((B // 16 - 1) - k2), 16)
            v = off_v[col]
            cm = plsc.cummax(lax.rev(-v, (0,)))
            sfx = lax.rev(-cm, (0,))
            out = jnp.minimum(sfx, carry)
            off_v[col] = out
            return jnp.full((16,), 1, jnp.int32) * jnp.min(out)

        lax.fori_loop(0, B // 16, suf_body,
                      jnp.full((16,), n, jnp.int32))

        base_graph = wid * GPT
        nlim = jnp.full((16,), n - 1, jnp.int32)

        def build_idx(b, idxbuf):
            vec = off_v[pl.ds(16 * (b // 16), 16)]
            sval = jnp.sum(jnp.where(iota == b % 16, vec, 0))
            start = jnp.full((16,), 1, jnp.int32) * sval
            for kk in range(ROWS // 16):
                idxbuf[pl.ds(16 * kk, 16)] = jnp.minimum(
                    start + iota + 16 * kk, nlim)

        def gather_start(idxbuf, buf, sem):
            pltpu.make_async_copy(emb_hbm.at[idxbuf], buf, sem).start()

        def gather_wait(idxbuf, buf, sem):
            pltpu.make_async_copy(emb_hbm.at[idxbuf], buf, sem).wait()

        def out_start(g, st, sem):
            pltpu.make_async_copy(
                st, out_hbm.at[pl.ds(base_graph + g, 1)], sem).start()

        def out_wait(st, sem):
            pltpu.make_async_copy(
                st, out_hbm.at[pl.ds(base_graph, 1)], sem).wait()

        build_idx(base_graph, idx0)
        gather_start(idx0, buf0, sg0)

        def body(i2, carry):
            g = 2 * i2
            build_idx(base_graph + g + 1, idx1)
            gather_start(idx1, buf1, sg1)
            gather_wait(idx0, buf0, sg0)

            @pl.when(i2 > 0)
            def _():
                out_wait(st0, so0)

            _pool_block(buf0, st0)
            out_start(g, st0, so0)

            @pl.when(i2 < GPT // 2 - 1)
            def _():
                build_idx(base_graph + g + 2, idx0)
                gather_start(idx0, buf0, sg0)

            gather_wait(idx1, buf1, sg1)

            @pl.when(i2 > 0)
            def _():
                out_wait(st1, so1)

            _pool_block(buf1, st1)
            out_start(g + 1, st1, so1)
            return carry

        lax.fori_loop(0, GPT // 2, body, 0)
        out_wait(st0, so0)
        out_wait(st1, so1)

    return k(emb, batch)


def _mlp_body(x_ref, w1_ref, b1_ref, g1_ref, bb1_ref,
              w2_ref, b2_ref, g2_ref, bb2_ref, o_ref):
    x = x_ref[...]
    h1 = jnp.dot(x, w1_ref[0], preferred_element_type=jnp.float32)
    h1 = h1 + b1_ref[0]
    mu = jnp.mean(h1, axis=-1, keepdims=True)
    var = jnp.mean((h1 - mu) ** 2, axis=-1, keepdims=True)
    h1 = (h1 - mu) / jnp.sqrt(var + 1e-5) * g1_ref[...] + bb1_ref[...]
    h1 = jnp.maximum(h1, 0.0)
    o = jnp.dot(h1, w2_ref[0], preferred_element_type=jnp.float32)
    o = o + b2_ref[0]
    mu2 = jnp.mean(o, axis=-1, keepdims=True)
    var2 = jnp.mean((o - mu2) ** 2, axis=-1, keepdims=True)
    o = (o - mu2) / jnp.sqrt(var2 + 1e-5) * g2_ref[...] + bb2_ref[...]
    o_ref[...] = jnp.maximum(o, 0.0)


def _mlp(comb, W1, b1, ln1_g, ln1_b, W2, b2, ln2_g, ln2_b):
    Bblk = 256
    grid = (H, B // Bblk)
    return pl.pallas_call(
        _mlp_body,
        grid=grid,
        in_specs=[
            pl.BlockSpec((Bblk, 3 * D), lambda h, i: (i, h)),
            pl.BlockSpec((1, 3 * D, 2 * HD), lambda h, i: (h, 0, 0)),
            pl.BlockSpec((1, 1, 2 * HD), lambda h, i: (h, 0, 0)),
            pl.BlockSpec((1, 2 * HD), lambda h, i: (0, 0)),
            pl.BlockSpec((1, 2 * HD), lambda h, i: (0, 0)),
            pl.BlockSpec((1, 2 * HD, HD), lambda h, i: (h, 0, 0)),
            pl.BlockSpec((1, 1, HD), lambda h, i: (h, 0, 0)),
            pl.BlockSpec((1, HD), lambda h, i: (0, 0)),
            pl.BlockSpec((1, HD), lambda h, i: (0, 0)),
        ],
        out_specs=pl.BlockSpec((Bblk, HD), lambda h, i: (i, h)),
        out_shape=jax.ShapeDtypeStruct((B, H * HD), jnp.float32),
    )(comb, W1, b1.reshape(H, 1, 2 * HD), ln1_g.reshape(1, -1),
      ln1_b.reshape(1, -1), W2, b2.reshape(H, 1, HD),
      ln2_g.reshape(1, -1), ln2_b.reshape(1, -1))


def kernel(node_embeddings, edge_index, batch, W1, b1, ln1_g, ln1_b,
           W2, b2, ln2_g, ln2_b):
    del edge_index
    comb = _sc_gather_pool(node_embeddings, batch)
    out = _mlp(comb, W1, b1, ln1_g, ln1_b, W2, b2, ln2_g, ln2_b)
    return out.reshape(B, H, HD)


# speedup vs baseline: 9.2101x; 1.0732x over previous
import functools

import jax
import jax.numpy as jnp
from jax import lax
from jax.experimental import pallas as pl
from jax.experimental.pallas import tpu as pltpu
from jax.experimental.pallas import tpu_sc as plsc

B = 2048
H = 8
D = 128
HD = 128
SUB = 16
ROWS = H * SUB

NC = 2
NS = 16
NW = NC * NS
GPT = B // NW


def _pool_block(buf, stage):
    def hbody(h, carry):
        row0 = SUB * h
        for c in range(D // 16):
            col = pl.ds(16 * c, 16)
            acc = [buf[row0 + r, col] for r in range(4)]
            mx = list(acc)
            for r in range(4, SUB):
                v = buf[row0 + r, col]
                acc[r % 4] = acc[r % 4] + v
                mx[r % 4] = jnp.maximum(mx[r % 4], v)
            s = (acc[0] + acc[1]) + (acc[2] + acc[3])
            m = jnp.maximum(jnp.maximum(mx[0], mx[1]),
                            jnp.maximum(mx[2], mx[3]))
            base = h * (3 * D) + 16 * c
            stage[0, pl.ds(base, 16)] = s * (1.0 / SUB)
            stage[0, pl.ds(base + D, 16)] = m
            stage[0, pl.ds(base + 2 * D, 16)] = s
        return carry

    lax.fori_loop(0, H, hbody, 0)


def _sc_gather_pool(emb, batch):
    n = emb.shape[0]
    scan = n // NS
    nvec = scan // 16
    mesh = plsc.VectorSubcoreMesh(
        core_axis_name="c", subcore_axis_name="s",
        num_cores=NC, num_subcores=NS)

    @functools.partial(
        pl.kernel,
        out_type=jax.ShapeDtypeStruct((B, 3 * H * D), jnp.float32),
        mesh=mesh,
        compiler_params=pltpu.CompilerParams(needs_layout_passes=False),
        scratch_types=[
            pltpu.VMEM((16 + scan,), jnp.int32),
            pltpu.VMEM((B,), jnp.int32),
            pltpu.VMEM((NS, B), jnp.int32),
            pltpu.VMEM((B,), jnp.int32),
            pltpu.VMEM((ROWS,), jnp.int32),
            pltpu.VMEM((ROWS,), jnp.int32),
            pltpu.VMEM((ROWS, D), jnp.float32),
            pltpu.VMEM((ROWS, D), jnp.float32),
            pltpu.VMEM((1, 3 * H * D), jnp.float32),
            pltpu.VMEM((1, 3 * H * D), jnp.float32),
            pltpu.VMEM_SHARED((NS, B), jnp.int32),
            pltpu.SemaphoreType.DMA,
            pltpu.SemaphoreType.DMA,
            pltpu.SemaphoreType.DMA,
            pltpu.SemaphoreType.DMA,
        ],
    )
    def k(emb_hbm, batch_hbm, out_hbm, chunk_v, part_v, allp_v, off_v,
          idx0, idx1, buf0, buf1, st0, st1, spmem, sg0, sg1, so0, so1):
        cid = lax.axis_index("c")
        sid = lax.axis_index("s")
        wid = sid * NC + cid
        iota = lax.iota(jnp.int32, 16)

        cstart = sid * scan
        pltpu.sync_copy(batch_hbm.at[pl.ds(cstart, scan)],
                        chunk_v.at[pl.ds(16, scan)])

        @pl.when(sid == 0)
        def _():
            chunk_v[pl.ds(0, 16)] = jnp.full((16,), -1, jnp.int32)

        @pl.when(sid > 0)
        def _():
            pltpu.sync_copy(batch_hbm.at[pl.ds(cstart - 16, 16)],
                            chunk_v.at[pl.ds(0, 16)])

        def initp(kk, carry):
            part_v[pl.ds(16 * kk, 16)] = jnp.full((16,), n, jnp.int32)
            return carry

        lax.fori_loop(0, B // 16, initp, 0)

        def scan_body(i, carry):
            base = 16 * i
            cur = chunk_v[pl.ds(16 + base, 16)]
            prev = chunk_v[pl.ds(15 + base, 16)]
            pos = jnp.full((16,), cstart + base, jnp.int32) + iota
            plsc.store_scatter(part_v, [cur], pos, mask=cur != prev)
            return carry

        lax.fori_loop(0, nvec, scan_body, 0)

        pltpu.sync_copy(part_v, spmem.at[sid])
        plsc.subcore_barrier()
        pltpu.sync_copy(spmem, allp_v)

        def comb_body(kk, carry):
            col = pl.ds(16 * kk, 16)
            m = allp_v[0, col]
            for s2 in range(1, NS):
                m = jnp.minimum(m, allp_v[s2, col])
            off_v[col] = m
            return carry

        lax.fori_loop(0, B // 16, comb_body, 0)

        def suf_body(k2, carry):
            col = pl.ds(16 * ((B // 16 - 1) - k2), 16)
            v = off_v[col]
            cm = plsc.cummax(lax.rev(-v, (0,)))
            sfx = lax.rev(-cm, (0,))
            out = jnp.minimum(sfx, carry)
            off_v[col] = out
            return jnp.full((16,), 1, jnp.int32) * jnp.min(out)

        lax.fori_loop(0, B // 16, suf_body,
                      jnp.full((16,), n, jnp.int32))

        base_graph = wid * GPT
        nlim = jnp.full((16,), n - 1, jnp.int32)

        def build_idx(b, idxbuf):
            vec = off_v[pl.ds(16 * (b // 16), 16)]
            sval = jnp.sum(jnp.where(iota == b % 16, vec, 0))
            start = jnp.full((16,), 1, jnp.int32) * sval
            for kk in range(ROWS // 16):
                idxbuf[pl.ds(16 * kk, 16)] = jnp.minimum(
                    start + iota + 16 * kk, nlim)

        def gather_start(idxbuf, buf, sem):
            pltpu.make_async_copy(emb_hbm.at[idxbuf], buf, sem).start()

        def gather_wait(idxbuf, buf, sem):
            pltpu.make_async_copy(emb_hbm.at[idxbuf], buf, sem).wait()

        def out_start(g, st, sem):
            pltpu.make_async_copy(
                st, out_hbm.at[pl.ds(base_graph + g, 1)], sem).start()

        def out_wait(st, sem):
            pltpu.make_async_copy(
                st, out_hbm.at[pl.ds(base_graph, 1)], sem).wait()

        build_idx(base_graph, idx0)
        gather_start(idx0, buf0, sg0)

        def body(i2, carry):
            g = 2 * i2
            build_idx(base_graph + g + 1, idx1)
            gather_start(idx1, buf1, sg1)
            gather_wait(idx0, buf0, sg0)

            @pl.when(i2 > 0)
            def _():
                out_wait(st0, so0)

            _pool_block(buf0, st0)
            out_start(g, st0, so0)

            @pl.when(i2 < GPT // 2 - 1)
            def _():
                build_idx(base_graph + g + 2, idx0)
                gather_start(idx0, buf0, sg0)

            gather_wait(idx1, buf1, sg1)

            @pl.when(i2 > 0)
            def _():
                out_wait(st1, so1)

            _pool_block(buf1, st1)
            out_start(g + 1, st1, so1)
            return carry

        lax.fori_loop(0, GPT // 2, body, 0)
        out_wait(st0, so0)
        out_wait(st1, so1)

    return k(emb, batch)


def _mlp_body(x_ref, w1_ref, b1_ref, g1_ref, bb1_ref,
              w2_ref, b2_ref, g2_ref, bb2_ref, o_ref):
    x = x_ref[...]
    h1 = jnp.dot(x, w1_ref[0], preferred_element_type=jnp.float32)
    h1 = h1 + b1_ref[0]
    mu = jnp.mean(h1, axis=-1, keepdims=True)
    var = jnp.mean((h1 - mu) ** 2, axis=-1, keepdims=True)
    h1 = (h1 - mu) / jnp.sqrt(var + 1e-5) * g1_ref[...] + bb1_ref[...]
    h1 = jnp.maximum(h1, 0.0)
    o = jnp.dot(h1, w2_ref[0], preferred_element_type=jnp.float32)
    o = o + b2_ref[0]
    mu2 = jnp.mean(o, axis=-1, keepdims=True)
    var2 = jnp.mean((o - mu2) ** 2, axis=-1, keepdims=True)
    o = (o - mu2) / jnp.sqrt(var2 + 1e-5) * g2_ref[...] + bb2_ref[...]
    o_ref[...] = jnp.maximum(o, 0.0)


def _mlp(comb, W1, b1, ln1_g, ln1_b, W2, b2, ln2_g, ln2_b):
    Bblk = 256
    grid = (H, B // Bblk)
    return pl.pallas_call(
        _mlp_body,
        grid=grid,
        in_specs=[
            pl.BlockSpec((Bblk, 3 * D), lambda h, i: (i, h)),
            pl.BlockSpec((1, 3 * D, 2 * HD), lambda h, i: (h, 0, 0)),
            pl.BlockSpec((1, 1, 2 * HD), lambda h, i: (h, 0, 0)),
            pl.BlockSpec((1, 2 * HD), lambda h, i: (0, 0)),
            pl.BlockSpec((1, 2 * HD), lambda h, i: (0, 0)),
            pl.BlockSpec((1, 2 * HD, HD), lambda h, i: (h, 0, 0)),
            pl.BlockSpec((1, 1, HD), lambda h, i: (h, 0, 0)),
            pl.BlockSpec((1, HD), lambda h, i: (0, 0)),
            pl.BlockSpec((1, HD), lambda h, i: (0, 0)),
        ],
        out_specs=pl.BlockSpec((Bblk, HD), lambda h, i: (i, h)),
        out_shape=jax.ShapeDtypeStruct((B, H * HD), jnp.float32),
    )(comb, W1, b1.reshape(H, 1, 2 * HD), ln1_g.reshape(1, -1),
      ln1_b.reshape(1, -1), W2, b2.reshape(H, 1, HD),
      ln2_g.reshape(1, -1), ln2_b.reshape(1, -1))


def kernel(node_embeddings, edge_index, batch, W1, b1, ln1_g, ln1_b,
           W2, b2, ln2_g, ln2_b):
    del edge_index
    comb = _sc_gather_pool(node_embeddings, batch)
    out = _mlp(comb, W1, b1, ln1_g, ln1_b, W2, b2, ln2_g, ln2_b)
    return out.reshape(B, H, HD)


# speedup vs baseline: 9.2169x; 1.0007x over previous
import functools

import jax
import jax.numpy as jnp
from jax import lax
from jax.experimental import pallas as pl
from jax.experimental.pallas import tpu as pltpu
from jax.experimental.pallas import tpu_sc as plsc

B = 2048
H = 8
D = 128
HD = 128
SUB = 16
ROWS = H * SUB

NC = 2
NS = 16
NW = NC * NS
GPT = B // NW


def _pool_block(buf, stage):
    def hbody(h, carry):
        base0 = SUB * h * D
        for c in range(D // 16):
            cb = 16 * c
            acc = [buf[pl.ds(base0 + r * D + cb, 16)] for r in range(4)]
            mx = list(acc)
            for r in range(4, SUB):
                v = buf[pl.ds(base0 + r * D + cb, 16)]
                acc[r % 4] = acc[r % 4] + v
                mx[r % 4] = jnp.maximum(mx[r % 4], v)
            s = (acc[0] + acc[1]) + (acc[2] + acc[3])
            m = jnp.maximum(jnp.maximum(mx[0], mx[1]),
                            jnp.maximum(mx[2], mx[3]))
            base = h * (3 * D) + 16 * c
            stage[0, pl.ds(base, 16)] = s * (1.0 / SUB)
            stage[0, pl.ds(base + D, 16)] = m
            stage[0, pl.ds(base + 2 * D, 16)] = s
        return carry

    lax.fori_loop(0, H, hbody, 0)


def _sc_gather_pool(emb, batch):
    n = emb.shape[0]
    emb_flat = emb.reshape(-1)
    scan = n // NS
    nvec = scan // 16
    mesh = plsc.VectorSubcoreMesh(
        core_axis_name="c", subcore_axis_name="s",
        num_cores=NC, num_subcores=NS)

    @functools.partial(
        pl.kernel,
        out_type=jax.ShapeDtypeStruct((B, 3 * H * D), jnp.float32),
        mesh=mesh,
        compiler_params=pltpu.CompilerParams(needs_layout_passes=False),
        scratch_types=[
            pltpu.VMEM((16 + scan,), jnp.int32),
            pltpu.VMEM((B,), jnp.int32),
            pltpu.VMEM((NS, B), jnp.int32),
            pltpu.VMEM((B,), jnp.int32),
            pltpu.VMEM((ROWS * D,), jnp.float32),
            pltpu.VMEM((ROWS * D,), jnp.float32),
            pltpu.VMEM((1, 3 * H * D), jnp.float32),
            pltpu.VMEM((1, 3 * H * D), jnp.float32),
            pltpu.VMEM_SHARED((NS, B), jnp.int32),
            pltpu.SemaphoreType.DMA,
            pltpu.SemaphoreType.DMA,
            pltpu.SemaphoreType.DMA,
            pltpu.SemaphoreType.DMA,
        ],
    )
    def k(emb_hbm, batch_hbm, out_hbm, chunk_v, part_v, allp_v, off_v,
          buf0, buf1, st0, st1, spmem, sg0, sg1, so0, so1):
        cid = lax.axis_index("c")
        sid = lax.axis_index("s")
        wid = sid * NC + cid
        iota = lax.iota(jnp.int32, 16)

        cstart = sid * scan
        pltpu.sync_copy(batch_hbm.at[pl.ds(cstart, scan)],
                        chunk_v.at[pl.ds(16, scan)])

        @pl.when(sid == 0)
        def _():
            chunk_v[pl.ds(0, 16)] = jnp.full((16,), -1, jnp.int32)

        @pl.when(sid > 0)
        def _():
            pltpu.sync_copy(batch_hbm.at[pl.ds(cstart - 16, 16)],
                            chunk_v.at[pl.ds(0, 16)])

        def initp(kk, carry):
            part_v[pl.ds(16 * kk, 16)] = jnp.full((16,), n, jnp.int32)
            return carry

        lax.fori_loop(0, B // 16, initp, 0)

        def scan_body(i, carry):
            base = 16 * i
            cur = chunk_v[pl.ds(16 + base, 16)]
            prev = chunk_v[pl.ds(15 + base, 16)]
            pos = jnp.full((16,), cstart + base, jnp.int32) + iota
            plsc.store_scatter(part_v, [cur], pos, mask=cur != prev)
            return carry

        lax.fori_loop(0, nvec, scan_body, 0)

        pltpu.sync_copy(part_v, spmem.at[sid])
        plsc.subcore_barrier()
        pltpu.sync_copy(spmem, allp_v)

        def comb_body(kk, carry):
            col = pl.ds(16 * kk, 16)
            m = allp_v[0, col]
            for s2 in range(1, NS):
                m = jnp.minimum(m, allp_v[s2, col])
            off_v[col] = m
            return carry

        lax.fori_loop(0, B // 16, comb_body, 0)

        def suf_body(k2, carry):
            col = pl.ds(16 * ((B // 16 - 1) - k2), 16)
            v = off_v[col]
            cm = plsc.cummax(lax.rev(-v, (0,)))
            sfx = lax.rev(-cm, (0,))
            out = jnp.minimum(sfx, carry)
            off_v[col] = out
            return jnp.full((16,), 1, jnp.int32) * jnp.min(out)

        lax.fori_loop(0, B // 16, suf_body,
                      jnp.full((16,), n, jnp.int32))

        base_graph = wid * GPT

        def gather_start(b, buf, sem):
            vec = off_v[pl.ds(16 * (b // 16), 16)]
            sval = jnp.sum(jnp.where(iota == b % 16, vec, 0))

            @pl.when(sval <= n - ROWS)
            def _():
                pltpu.make_async_copy(
                    emb_hbm.at[pl.ds(sval * D, ROWS * D)], buf, sem).start()

            @pl.when(sval > n - ROWS)
            def _():
                def rowcopy(r, carry):
                    src = jnp.minimum(sval + r, n - 1) * D
                    pltpu.make_async_copy(
                        emb_hbm.at[pl.ds(src, D)],
                        buf.at[pl.ds(r * D, D)], sem).start()
                    return carry

                lax.fori_loop(0, ROWS, rowcopy, 0)

        def gather_wait(buf, sem):
            pltpu.make_async_copy(
                emb_hbm.at[pl.ds(0, ROWS * D)], buf, sem).wait()

        def out_start(g, st, sem):
            pltpu.make_async_copy(
                st, out_hbm.at[pl.ds(base_graph + g, 1)], sem).start()

        def out_wait(st, sem):
            pltpu.make_async_copy(
                st, out_hbm.at[pl.ds(base_graph, 1)], sem).wait()

        gather_start(base_graph, buf0, sg0)

        def body(i2, carry):
            g = 2 * i2
            gather_start(base_graph + g + 1, buf1, sg1)
            gather_wait(buf0, sg0)

            @pl.when(i2 > 0)
            def _():
                out_wait(st0, so0)

            _pool_block(buf0, st0)
            out_start(g, st0, so0)

            @pl.when(i2 < GPT // 2 - 1)
            def _():
                gather_start(base_graph + g + 2, buf0, sg0)

            gather_wait(buf1, sg1)

            @pl.when(i2 > 0)
            def _():
                out_wait(st1, so1)

            _pool_block(buf1, st1)
            out_start(g + 1, st1, so1)
            return carry

        lax.fori_loop(0, GPT // 2, body, 0)
        out_wait(st0, so0)
        out_wait(st1, so1)

    return k(emb_flat, batch)


def _mlp_body(x_ref, w1_ref, b1_ref, g1_ref, bb1_ref,
              w2_ref, b2_ref, g2_ref, bb2_ref, o_ref):
    x = x_ref[...]
    h1 = jnp.dot(x, w1_ref[0], preferred_element_type=jnp.float32)
    h1 = h1 + b1_ref[0]
    mu = jnp.mean(h1, axis=-1, keepdims=True)
    var = jnp.mean((h1 - mu) ** 2, axis=-1, keepdims=True)
    h1 = (h1 - mu) / jnp.sqrt(var + 1e-5) * g1_ref[...] + bb1_ref[...]
    h1 = jnp.maximum(h1, 0.0)
    o = jnp.dot(h1, w2_ref[0], preferred_element_type=jnp.float32)
    o = o + b2_ref[0]
    mu2 = jnp.mean(o, axis=-1, keepdims=True)
    var2 = jnp.mean((o - mu2) ** 2, axis=-1, keepdims=True)
    o = (o - mu2) / jnp.sqrt(var2 + 1e-5) * g2_ref[...] + bb2_ref[...]
    o_ref[...] = jnp.maximum(o, 0.0)


def _mlp(comb, W1, b1, ln1_g, ln1_b, W2, b2, ln2_g, ln2_b):
    Bblk = 256
    grid = (H, B // Bblk)
    return pl.pallas_call(
        _mlp_body,
        grid=grid,
        in_specs=[
            pl.BlockSpec((Bblk, 3 * D), lambda h, i: (i, h)),
            pl.BlockSpec((1, 3 * D, 2 * HD), lambda h, i: (h, 0, 0)),
            pl.BlockSpec((1, 1, 2 * HD), lambda h, i: (h, 0, 0)),
            pl.BlockSpec((1, 2 * HD), lambda h, i: (0, 0)),
            pl.BlockSpec((1, 2 * HD), lambda h, i: (0, 0)),
            pl.BlockSpec((1, 2 * HD, HD), lambda h, i: (h, 0, 0)),
            pl.BlockSpec((1, 1, HD), lambda h, i: (h, 0, 0)),
            pl.BlockSpec((1, HD), lambda h, i: (0, 0)),
            pl.BlockSpec((1, HD), lambda h, i: (0, 0)),
        ],
        out_specs=pl.BlockSpec((Bblk, HD), lambda h, i: (i, h)),
        out_shape=jax.ShapeDtypeStruct((B, H * HD), jnp.float32),
    )(comb, W1, b1.reshape(H, 1, 2 * HD), ln1_g.reshape(1, -1),
      ln1_b.reshape(1, -1), W2, b2.reshape(H, 1, HD),
      ln2_g.reshape(1, -1), ln2_b.reshape(1, -1))


def kernel(node_embeddings, edge_index, batch, W1, b1, ln1_g, ln1_b,
           W2, b2, ln2_g, ln2_b):
    del edge_index
    comb = _sc_gather_pool(node_embeddings, batch)
    out = _mlp(comb, W1, b1, ln1_g, ln1_b, W2, b2, ln2_g, ln2_b)
    return out.reshape(B, H, HD)
